# baseline (speedup 1.0000x reference)
import functools

import jax
import jax.numpy as jnp
from jax import lax
from jax.experimental import pallas as pl
from jax.experimental.pallas import tpu as pltpu
from jax.experimental.pallas import tpu_sc as plsc

DIM = 768
HID = 2048
E = 8
K = 2
T = 2048
PAIRS = T * K
TM = 512
TMSH = 9
NTILES = 16
PADDED = NTILES * TM
LANES = 128

NW = 32
PP = PAIRS // NW
TPW = T // NW
NCHUNK = PAIRS // 16
MYCH = PP // 16


def _gate_body(x_ref, wg_ref, idx_ref, w_ref):
    x = x_ref[...]
    lg = lax.dot_general(x, wg_ref[...], (((1,), (1,)), ((), ())),
                         preferred_element_type=jnp.float32)
    lane = lax.broadcasted_iota(jnp.int32, lg.shape, 1)
    neg = jnp.float32(-1e30)
    lg = jnp.where(lane < E, lg, neg)
    m1 = jnp.max(lg, axis=1, keepdims=True)
    i1 = jnp.min(jnp.where(lg == m1, lane, LANES), axis=1, keepdims=True)
    lg2 = jnp.where(lane == i1, neg, lg)
    m2 = jnp.max(lg2, axis=1, keepdims=True)
    i2 = jnp.min(jnp.where(lg2 == m2, lane, LANES), axis=1, keepdims=True)
    w_top = 1.0 / (1.0 + jnp.exp(m2 - m1))
    w_snd = 1.0 - w_top
    idx_ref[...] = jnp.where(lane == 0, i1, jnp.where(lane == 1, i2, 0))
    w_ref[...] = jnp.where(lane == 0, w_top, jnp.where(lane == 1, w_snd, 0.0))


def _gate(flat, wgp):
    return pl.pallas_call(
        _gate_body,
        grid=(1,),
        in_specs=[
            pl.BlockSpec((T, DIM), lambda i: (0, 0)),
            pl.BlockSpec((LANES, DIM), lambda i: (0, 0)),
        ],
        out_specs=[
            pl.BlockSpec((T, LANES), lambda i: (0, 0)),
            pl.BlockSpec((T, LANES), lambda i: (0, 0)),
        ],
        out_shape=[
            jax.ShapeDtypeStruct((T, LANES), jnp.int32),
            jax.ShapeDtypeStruct((T, LANES), jnp.float32),
        ],
    )(flat, wgp)


def _route_body(eall_hbm, x_hbm, xs_hbm, pos_hbm, te_hbm,
                eall_v, xrows_v, tok_v, dest_v, cnt_v, gbuf_v, te_v,
                sem1, sem2):
    wid = lax.axis_index("s") * 2 + lax.axis_index("c")
    base = wid * PP
    lane = lax.broadcasted_iota(jnp.int32, (16,), 0)
    ones = jnp.ones((16,), jnp.int32)

    pltpu.sync_copy(eall_hbm, eall_v)
    mych0 = wid * MYCH
    cnt_v[pl.ds(0, 16)] = jnp.zeros((16,), jnp.int32)

    def add_step(i, _):
        v = eall_v[pl.ds(i * 16, 16)]
        plsc.addupdate_scatter(cnt_v, [v], ones)
        return 0

    lax.fori_loop(0, mych0, add_step, 0)

    for c in range(MYCH):
        v = eall_v[pl.ds((mych0 + c) * 16, 16)]
        g = plsc.load_gather(cnt_v, [v])
        rank = jnp.zeros((16,), jnp.int32)
        for e in range(E):
            m = v == e
            cm = plsc.cumsum(m.astype(jnp.int32))
            rank = jnp.where(m, cm - 1, rank)
        dest_v[pl.ds(c * 16, 16)] = g + rank
        plsc.addupdate_scatter(cnt_v, [v], ones)
        tok_v[pl.ds(c * 16, 16)] = (base + c * 16 + lane) >> 1

    lax.fori_loop(mych0 + MYCH, NCHUNK, add_step, 0)
    counts = cnt_v[pl.ds(0, 16)]

    pc = ((counts + (TM - 1)) >> TMSH) << TMSH
    cs = plsc.cumsum(pc)
    po = cs - pc

    cs_s = [jnp.sum(jnp.where(lane == e, cs, 0)) for e in range(E)]
    for c in range(4):
        tj = (lane + c * 16) * TM
        acc = jnp.zeros((16,), jnp.int32)
        for e in range(E):
            acc = acc + (tj >= cs_s[e]).astype(jnp.int32)
        te_v[pl.ds(c * 16, 16)] = jnp.minimum(acc, E - 1)

    @pl.when(wid == 0)
    def _():
        pltpu.sync_copy(te_v, te_hbm)

    gbuf_v[pl.ds(0, 16)] = po
    for c in range(MYCH):
        v = eall_v[pl.ds((mych0 + c) * 16, 16)]
        dest_v[pl.ds(c * 16, 16)] = (dest_v[pl.ds(c * 16, 16)]
                                     + plsc.load_gather(gbuf_v, [v]))

    pltpu.sync_copy(dest_v, pos_hbm.at[pl.ds(base, PP)])

    pltpu.async_copy(x_hbm.at[tok_v], xrows_v, sem1).wait()
    pltpu.async_copy(xrows_v, xs_hbm.at[dest_v], sem2).wait()


def _route(e_all, flat):
    mesh = plsc.VectorSubcoreMesh(core_axis_name="c", subcore_axis_name="s")
    return pl.kernel(
        _route_body,
        mesh=mesh,
        out_type=[
            jax.ShapeDtypeStruct((PADDED, DIM), jnp.float32),
            jax.ShapeDtypeStruct((PAIRS,), jnp.int32),
            jax.ShapeDtypeStruct((64,), jnp.int32),
        ],
        scratch_types=[
            pltpu.VMEM((PAIRS,), jnp.int32),
            pltpu.VMEM((PP, DIM), jnp.float32),
            pltpu.VMEM((PP,), jnp.int32),
            pltpu.VMEM((PP,), jnp.int32),
            pltpu.VMEM((16,), jnp.int32),
            pltpu.VMEM((32,), jnp.int32),
            pltpu.VMEM((64,), jnp.int32),
            pltpu.SemaphoreType.DMA,
            pltpu.SemaphoreType.DMA,
        ],
        compiler_params=pltpu.CompilerParams(needs_layout_passes=False),
    )(e_all, flat)


def _gemm_body(te_ref, x_ref, w1_ref, w3_ref, w2_ref, o_ref):
    x = x_ref[...].astype(jnp.bfloat16)
    h1 = lax.dot_general(x, w1_ref[0].astype(jnp.bfloat16),
                         (((1,), (1,)), ((), ())),
                         preferred_element_type=jnp.float32)
    h3 = lax.dot_general(x, w3_ref[0].astype(jnp.bfloat16),
                         (((1,), (1,)), ((), ())),
                         preferred_element_type=jnp.float32)
    hid = ((h1 * jax.nn.sigmoid(h1)) * h3).astype(jnp.bfloat16)
    o_ref[...] = lax.dot_general(hid, w2_ref[0].astype(jnp.bfloat16),
                                 (((1,), (1,)), ((), ())),
                                 preferred_element_type=jnp.float32)


def _gemm(te, xs, W1, W3, W2):
    grid_spec = pltpu.PrefetchScalarGridSpec(
        num_scalar_prefetch=1,
        grid=(NTILES,),
        in_specs=[
            pl.BlockSpec((TM, DIM), lambda j, te_r: (j, 0)),
            pl.BlockSpec((1, HID, DIM), lambda j, te_r: (te_r[j], 0, 0)),
            pl.BlockSpec((1, HID, DIM), lambda j, te_r: (te_r[j], 0, 0)),
            pl.BlockSpec((1, DIM, HID), lambda j, te_r: (te_r[j], 0, 0)),
        ],
        out_specs=pl.BlockSpec((TM, DIM), lambda j, te_r: (j, 0)),
    )
    return pl.pallas_call(
        _gemm_body,
        grid_spec=grid_spec,
        out_shape=jax.ShapeDtypeStruct((PADDED, DIM), jnp.float32),
    )(te, xs, W1, W3, W2)


THALF = TPW // 2


def _combine_body(yp_hbm, pos_hbm, w_hbm, y_hbm,
                  posh_v, w_v, rows_v, y_v, sem1):
    wid = lax.axis_index("s") * 2 + lax.axis_index("c")
    pbase = wid * PP

    pltpu.sync_copy(w_hbm.at[pl.ds(pbase, PP)], w_v)
    for h in range(2):
        pltpu.sync_copy(pos_hbm.at[pl.ds(pbase + h * 2 * THALF, 2 * THALF)],
                        posh_v)
        pltpu.async_copy(yp_hbm.at[posh_v], rows_v, sem1).wait()

        def tok_step(i, _, h=h):
            base_w = h * 2 * THALF
            w0 = plsc.load_gather(
                w_v, [jnp.zeros((16,), jnp.int32) + base_w + 2 * i])
            w1 = plsc.load_gather(
                w_v, [jnp.zeros((16,), jnp.int32) + base_w + 2 * i + 1])
            for j in range(DIM // 16):
                r0 = rows_v[2 * i, pl.ds(j * 16, 16)]
                r1 = rows_v[2 * i + 1, pl.ds(j * 16, 16)]
                y_v[i, pl.ds(j * 16, 16)] = w0 * r0 + w1 * r1
            return 0

        lax.fori_loop(0, THALF, tok_step, 0)
        pltpu.sync_copy(y_v, y_hbm.at[pl.ds(wid * TPW + h * THALF, THALF)])


def _combine(ypart, pos, w_all):
    mesh = plsc.VectorSubcoreMesh(core_axis_name="c", subcore_axis_name="s")
    return pl.kernel(
        _combine_body,
        mesh=mesh,
        out_type=jax.ShapeDtypeStruct((T, DIM), jnp.float32),
        scratch_types=[
            pltpu.VMEM((2 * THALF,), jnp.int32),
            pltpu.VMEM((PP,), jnp.float32),
            pltpu.VMEM((2 * THALF, DIM), jnp.float32),
            pltpu.VMEM((THALF, DIM), jnp.float32),
            pltpu.SemaphoreType.DMA,
        ],
        compiler_params=pltpu.CompilerParams(needs_layout_passes=False),
    )(ypart, pos, w_all)


def kernel(x, Wg, W1, W2, W3):
    b, s, d = x.shape
    flat = x.reshape(-1, d)
    wgp = jnp.zeros((LANES, DIM), jnp.float32).at[:E].set(Wg)

    idx_out, w_out = _gate(flat, wgp)
    e_all = idx_out[:, :K].reshape(-1)
    w_all = w_out[:, :K].reshape(-1)

    xs, pos, te = _route(e_all, flat)
    ypart = _gemm(te[:NTILES], xs, W1, W3, W2)
    y = _combine(ypart, pos, w_all)
    return y.reshape(b, s, d)


# speedup vs baseline: 1.0996x; 1.0996x over previous
import functools

import jax
import jax.numpy as jnp
from jax import lax
from jax.experimental import pallas as pl
from jax.experimental.pallas import tpu as pltpu
from jax.experimental.pallas import tpu_sc as plsc

DIM = 768
HID = 2048
E = 8
K = 2
T = 2048
PAIRS = T * K
TM = 512
TMSH = 9
NTILES = 16
PADDED = NTILES * TM
LANES = 128

NW = 32
PP = PAIRS // NW
TPW = T // NW
NCHUNK = PAIRS // 16
MYCH = PP // 16


def _gate_body(x_ref, wg_ref, idx_ref, w_ref):
    x = x_ref[...]
    lg = lax.dot_general(x, wg_ref[...], (((1,), (1,)), ((), ())),
                         preferred_element_type=jnp.float32)
    lane = lax.broadcasted_iota(jnp.int32, lg.shape, 1)
    neg = jnp.float32(-1e30)
    lg = jnp.where(lane < E, lg, neg)
    m1 = jnp.max(lg, axis=1, keepdims=True)
    i1 = jnp.min(jnp.where(lg == m1, lane, LANES), axis=1, keepdims=True)
    lg2 = jnp.where(lane == i1, neg, lg)
    m2 = jnp.max(lg2, axis=1, keepdims=True)
    i2 = jnp.min(jnp.where(lg2 == m2, lane, LANES), axis=1, keepdims=True)
    w_top = 1.0 / (1.0 + jnp.exp(m2 - m1))
    w_snd = 1.0 - w_top
    idx_ref[...] = jnp.where(lane == 0, i1, jnp.where(lane == 1, i2, 0))
    w_ref[...] = jnp.where(lane == 0, w_top, jnp.where(lane == 1, w_snd, 0.0))


def _gate(flat, wgp):
    return pl.pallas_call(
        _gate_body,
        grid=(1,),
        in_specs=[
            pl.BlockSpec((T, DIM), lambda i: (0, 0)),
            pl.BlockSpec((LANES, DIM), lambda i: (0, 0)),
        ],
        out_specs=[
            pl.BlockSpec((T, LANES), lambda i: (0, 0)),
            pl.BlockSpec((T, LANES), lambda i: (0, 0)),
        ],
        out_shape=[
            jax.ShapeDtypeStruct((T, LANES), jnp.int32),
            jax.ShapeDtypeStruct((T, LANES), jnp.float32),
        ],
    )(flat, wgp)


def _route_body(eall_hbm, x_hbm, xs_hbm, pos_hbm, te_hbm,
                eall_v, xrows_v, tok_v, dest_v, cnt_v, gbuf_v, te_v,
                sem1, sem2):
    wid = lax.axis_index("s") * 2 + lax.axis_index("c")
    base = wid * PP
    lane = lax.broadcasted_iota(jnp.int32, (16,), 0)
    ones = jnp.ones((16,), jnp.int32)

    pltpu.sync_copy(eall_hbm, eall_v)
    mych0 = wid * MYCH
    cnt_v[pl.ds(0, 16)] = jnp.zeros((16,), jnp.int32)

    def add_step(i, _):
        v = eall_v[pl.ds(i * 16, 16)]
        plsc.addupdate_scatter(cnt_v, [v], ones)
        return 0

    lax.fori_loop(0, mych0, add_step, 0)

    for c in range(MYCH):
        v = eall_v[pl.ds((mych0 + c) * 16, 16)]
        g = plsc.load_gather(cnt_v, [v])
        rank = jnp.zeros((16,), jnp.int32)
        for e in range(E):
            m = v == e
            cm = plsc.cumsum(m.astype(jnp.int32))
            rank = jnp.where(m, cm - 1, rank)
        dest_v[pl.ds(c * 16, 16)] = g + rank
        plsc.addupdate_scatter(cnt_v, [v], ones)
        tok_v[pl.ds(c * 16, 16)] = (base + c * 16 + lane) >> 1

    lax.fori_loop(mych0 + MYCH, NCHUNK, add_step, 0)
    counts = cnt_v[pl.ds(0, 16)]

    pc = ((counts + (TM - 1)) >> TMSH) << TMSH
    cs = plsc.cumsum(pc)
    po = cs - pc

    cs_s = [jnp.sum(jnp.where(lane == e, cs, 0)) for e in range(E)]
    for c in range(4):
        tj = (lane + c * 16) * TM
        acc = jnp.zeros((16,), jnp.int32)
        for e in range(E):
            acc = acc + (tj >= cs_s[e]).astype(jnp.int32)
        te_v[pl.ds(c * 16, 16)] = acc

    @pl.when(wid == 0)
    def _():
        pltpu.sync_copy(te_v, te_hbm)

    gbuf_v[pl.ds(0, 16)] = po
    for c in range(MYCH):
        v = eall_v[pl.ds((mych0 + c) * 16, 16)]
        dest_v[pl.ds(c * 16, 16)] = (dest_v[pl.ds(c * 16, 16)]
                                     + plsc.load_gather(gbuf_v, [v]))

    pltpu.sync_copy(dest_v, pos_hbm.at[pl.ds(base, PP)])

    pltpu.async_copy(x_hbm.at[tok_v], xrows_v, sem1).wait()
    pltpu.async_copy(xrows_v, xs_hbm.at[dest_v], sem2).wait()


def _route(e_all, flat):
    mesh = plsc.VectorSubcoreMesh(core_axis_name="c", subcore_axis_name="s")
    return pl.kernel(
        _route_body,
        mesh=mesh,
        out_type=[
            jax.ShapeDtypeStruct((PADDED, DIM), jnp.float32),
            jax.ShapeDtypeStruct((PAIRS,), jnp.int32),
            jax.ShapeDtypeStruct((64,), jnp.int32),
        ],
        scratch_types=[
            pltpu.VMEM((PAIRS,), jnp.int32),
            pltpu.VMEM((PP, DIM), jnp.float32),
            pltpu.VMEM((PP,), jnp.int32),
            pltpu.VMEM((PP,), jnp.int32),
            pltpu.VMEM((16,), jnp.int32),
            pltpu.VMEM((32,), jnp.int32),
            pltpu.VMEM((64,), jnp.int32),
            pltpu.SemaphoreType.DMA,
            pltpu.SemaphoreType.DMA,
        ],
        compiler_params=pltpu.CompilerParams(needs_layout_passes=False),
    )(e_all, flat)


def _gemm_body(te_ref, x_ref, w1_ref, w3_ref, w2_ref, o_ref):
    j = pl.program_id(0)

    @pl.when(te_ref[j] < E)
    def _():
        x = x_ref[...].astype(jnp.bfloat16)
        h1 = lax.dot_general(x, w1_ref[0].astype(jnp.bfloat16),
                             (((1,), (1,)), ((), ())),
                             preferred_element_type=jnp.float32)
        h3 = lax.dot_general(x, w3_ref[0].astype(jnp.bfloat16),
                             (((1,), (1,)), ((), ())),
                             preferred_element_type=jnp.float32)
        hid = ((h1 * jax.nn.sigmoid(h1)) * h3).astype(jnp.bfloat16)
        o_ref[...] = lax.dot_general(hid, w2_ref[0].astype(jnp.bfloat16),
                                     (((1,), (1,)), ((), ())),
                                     preferred_element_type=jnp.float32)


def _gemm(te, xs, W1, W3, W2):
    grid_spec = pltpu.PrefetchScalarGridSpec(
        num_scalar_prefetch=1,
        grid=(NTILES,),
        in_specs=[
            pl.BlockSpec((TM, DIM), lambda j, te_r: (j, 0)),
            pl.BlockSpec((1, HID, DIM),
                         lambda j, te_r: (jnp.minimum(te_r[j], E - 1), 0, 0)),
            pl.BlockSpec((1, HID, DIM),
                         lambda j, te_r: (jnp.minimum(te_r[j], E - 1), 0, 0)),
            pl.BlockSpec((1, DIM, HID),
                         lambda j, te_r: (jnp.minimum(te_r[j], E - 1), 0, 0)),
        ],
        out_specs=pl.BlockSpec((TM, DIM), lambda j, te_r: (j, 0)),
    )
    return pl.pallas_call(
        _gemm_body,
        grid_spec=grid_spec,
        out_shape=jax.ShapeDtypeStruct((PADDED, DIM), jnp.float32),
    )(te, xs, W1, W3, W2)


THALF = TPW // 2


def _combine_body(yp_hbm, pos_hbm, w_hbm, y_hbm,
                  posh_v, w_v, rows_v, y_v, sem1):
    wid = lax.axis_index("s") * 2 + lax.axis_index("c")
    pbase = wid * PP

    pltpu.sync_copy(w_hbm.at[pl.ds(pbase, PP)], w_v)
    for h in range(2):
        pltpu.sync_copy(pos_hbm.at[pl.ds(pbase + h * 2 * THALF, 2 * THALF)],
                        posh_v)
        pltpu.async_copy(yp_hbm.at[posh_v], rows_v, sem1).wait()

        def tok_step(i, _, h=h):
            base_w = h * 2 * THALF
            w0 = plsc.load_gather(
                w_v, [jnp.zeros((16,), jnp.int32) + base_w + 2 * i])
            w1 = plsc.load_gather(
                w_v, [jnp.zeros((16,), jnp.int32) + base_w + 2 * i + 1])
            for j in range(DIM // 16):
                r0 = rows_v[2 * i, pl.ds(j * 16, 16)]
                r1 = rows_v[2 * i + 1, pl.ds(j * 16, 16)]
                y_v[i, pl.ds(j * 16, 16)] = w0 * r0 + w1 * r1
            return 0

        lax.fori_loop(0, THALF, tok_step, 0)
        pltpu.sync_copy(y_v, y_hbm.at[pl.ds(wid * TPW + h * THALF, THALF)])


def _combine(ypart, pos, w_all):
    mesh = plsc.VectorSubcoreMesh(core_axis_name="c", subcore_axis_name="s")
    return pl.kernel(
        _combine_body,
        mesh=mesh,
        out_type=jax.ShapeDtypeStruct((T, DIM), jnp.float32),
        scratch_types=[
            pltpu.VMEM((2 * THALF,), jnp.int32),
            pltpu.VMEM((PP,), jnp.float32),
            pltpu.VMEM((2 * THALF, DIM), jnp.float32),
            pltpu.VMEM((THALF, DIM), jnp.float32),
            pltpu.SemaphoreType.DMA,
        ],
        compiler_params=pltpu.CompilerParams(needs_layout_passes=False),
    )(ypart, pos, w_all)


def kernel(x, Wg, W1, W2, W3):
    b, s, d = x.shape
    flat = x.reshape(-1, d)
    wgp = jnp.zeros((LANES, DIM), jnp.float32).at[:E].set(Wg)

    idx_out, w_out = _gate(flat, wgp)
    e_all = idx_out[:, :K].reshape(-1)
    w_all = w_out[:, :K].reshape(-1)

    xs, pos, te = _route(e_all, flat)
    ypart = _gemm(te[:NTILES], xs, W1, W3, W2)
    y = _combine(ypart, pos, w_all)
    return y.reshape(b, s, d)


# speedup vs baseline: 1.1014x; 1.0016x over previous
import functools

import jax
import jax.numpy as jnp
from jax import lax
from jax.experimental import pallas as pl
from jax.experimental.pallas import tpu as pltpu
from jax.experimental.pallas import tpu_sc as plsc

DIM = 768
HID = 2048
E = 8
K = 2
T = 2048
PAIRS = T * K
TM = 512
TMSH = 9
NTILES = 16
PADDED = NTILES * TM
LANES = 128

NW = 32
PP = PAIRS // NW
TPW = T // NW
NCHUNK = PAIRS // 16
MYCH = PP // 16


def _gate_body(x_ref, wg_ref, idx_ref, w_ref):
    x = x_ref[...]
    lg = lax.dot_general(x, wg_ref[...], (((1,), (1,)), ((), ())),
                         preferred_element_type=jnp.float32)
    lane = lax.broadcasted_iota(jnp.int32, lg.shape, 1)
    neg = jnp.float32(-1e30)
    lg = jnp.where(lane < E, lg, neg)
    m1 = jnp.max(lg, axis=1, keepdims=True)
    i1 = jnp.min(jnp.where(lg == m1, lane, LANES), axis=1, keepdims=True)
    lg2 = jnp.where(lane == i1, neg, lg)
    m2 = jnp.max(lg2, axis=1, keepdims=True)
    i2 = jnp.min(jnp.where(lg2 == m2, lane, LANES), axis=1, keepdims=True)
    w_top = 1.0 / (1.0 + jnp.exp(m2 - m1))
    w_snd = 1.0 - w_top
    idx_ref[...] = jnp.where(lane == 0, i1, jnp.where(lane == 1, i2, 0))
    w_ref[...] = jnp.where(lane == 0, w_top, jnp.where(lane == 1, w_snd, 0.0))


def _gate(flat, wgp):
    return pl.pallas_call(
        _gate_body,
        grid=(1,),
        in_specs=[
            pl.BlockSpec((T, DIM), lambda i: (0, 0)),
            pl.BlockSpec((LANES, DIM), lambda i: (0, 0)),
        ],
        out_specs=[
            pl.BlockSpec((T, LANES), lambda i: (0, 0)),
            pl.BlockSpec((T, LANES), lambda i: (0, 0)),
        ],
        out_shape=[
            jax.ShapeDtypeStruct((T, LANES), jnp.int32),
            jax.ShapeDtypeStruct((T, LANES), jnp.float32),
        ],
    )(flat, wgp)


def _route_body(eall_hbm, x_hbm, xs_hbm, pos_hbm, te_hbm,
                eall_v, xrows_v, tok_v, dest_v, cnt_v, gbuf_v, te_v,
                sem1, sem2):
    wid = lax.axis_index("s") * 2 + lax.axis_index("c")
    base = wid * PP
    lane = lax.broadcasted_iota(jnp.int32, (16,), 0)
    ones = jnp.ones((16,), jnp.int32)

    pltpu.sync_copy(eall_hbm, eall_v)
    mych0 = wid * MYCH
    cnt_v[pl.ds(0, 16)] = jnp.zeros((16,), jnp.int32)

    def add_step(i, _):
        v = eall_v[pl.ds(i * 16, 16)]
        plsc.addupdate_scatter(cnt_v, [v], ones)
        return 0

    lax.fori_loop(0, mych0, add_step, 0)

    for c in range(MYCH):
        v = eall_v[pl.ds((mych0 + c) * 16, 16)]
        g = plsc.load_gather(cnt_v, [v])
        rank = jnp.zeros((16,), jnp.int32)
        for e in range(E):
            m = v == e
            cm = plsc.cumsum(m.astype(jnp.int32))
            rank = jnp.where(m, cm - 1, rank)
        dest_v[pl.ds(c * 16, 16)] = g + rank
        plsc.addupdate_scatter(cnt_v, [v], ones)
        tok_v[pl.ds(c * 16, 16)] = (base + c * 16 + lane) >> 1

    lax.fori_loop(mych0 + MYCH, NCHUNK, add_step, 0)
    counts = cnt_v[pl.ds(0, 16)]

    pc = ((counts + (TM - 1)) >> TMSH) << TMSH
    cs = plsc.cumsum(pc)
    po = cs - pc

    cs_s = [jnp.sum(jnp.where(lane == e, cs, 0)) for e in range(E)]
    for c in range(4):
        tj = (lane + c * 16) * TM
        acc = jnp.zeros((16,), jnp.int32)
        for e in range(E):
            acc = acc + (tj >= cs_s[e]).astype(jnp.int32)
        te_v[pl.ds(c * 16, 16)] = acc

    @pl.when(wid == 0)
    def _():
        pltpu.sync_copy(te_v, te_hbm)

    gbuf_v[pl.ds(0, 16)] = po
    for c in range(MYCH):
        v = eall_v[pl.ds((mych0 + c) * 16, 16)]
        dest_v[pl.ds(c * 16, 16)] = (dest_v[pl.ds(c * 16, 16)]
                                     + plsc.load_gather(gbuf_v, [v]))

    pltpu.sync_copy(dest_v, pos_hbm.at[pl.ds(base, PP)])

    pltpu.async_copy(x_hbm.at[tok_v], xrows_v, sem1).wait()
    pltpu.async_copy(xrows_v, xs_hbm.at[dest_v], sem2).wait()


def _route(e_all, flat):
    mesh = plsc.VectorSubcoreMesh(core_axis_name="c", subcore_axis_name="s")
    return pl.kernel(
        _route_body,
        mesh=mesh,
        out_type=[
            jax.ShapeDtypeStruct((PADDED, DIM), jnp.float32),
            jax.ShapeDtypeStruct((PAIRS,), jnp.int32),
            jax.ShapeDtypeStruct((64,), jnp.int32),
        ],
        scratch_types=[
            pltpu.VMEM((PAIRS,), jnp.int32),
            pltpu.VMEM((PP, DIM), jnp.float32),
            pltpu.VMEM((PP,), jnp.int32),
            pltpu.VMEM((PP,), jnp.int32),
            pltpu.VMEM((16,), jnp.int32),
            pltpu.VMEM((32,), jnp.int32),
            pltpu.VMEM((64,), jnp.int32),
            pltpu.SemaphoreType.DMA,
            pltpu.SemaphoreType.DMA,
        ],
        compiler_params=pltpu.CompilerParams(needs_layout_passes=False),
    )(e_all, flat)


def _gemm_body(te_ref, x_ref, w1_ref, w3_ref, w2_ref, o_ref):
    j = pl.program_id(0)

    @pl.when(te_ref[j] < E)
    def _():
        x = x_ref[...]
        h1 = lax.dot_general(x, w1_ref[0], (((1,), (1,)), ((), ())),
                             preferred_element_type=jnp.float32)
        h3 = lax.dot_general(x, w3_ref[0], (((1,), (1,)), ((), ())),
                             preferred_element_type=jnp.float32)
        hid = (h1 * jax.nn.sigmoid(h1)) * h3
        o_ref[...] = lax.dot_general(hid, w2_ref[0], (((1,), (1,)), ((), ())),
                                     preferred_element_type=jnp.float32)


def _gemm(te, xs, W1, W3, W2):
    grid_spec = pltpu.PrefetchScalarGridSpec(
        num_scalar_prefetch=1,
        grid=(NTILES,),
        in_specs=[
            pl.BlockSpec((TM, DIM), lambda j, te_r: (j, 0)),
            pl.BlockSpec((1, HID, DIM),
                         lambda j, te_r: (jnp.minimum(te_r[j], E - 1), 0, 0)),
            pl.BlockSpec((1, HID, DIM),
                         lambda j, te_r: (jnp.minimum(te_r[j], E - 1), 0, 0)),
            pl.BlockSpec((1, DIM, HID),
                         lambda j, te_r: (jnp.minimum(te_r[j], E - 1), 0, 0)),
        ],
        out_specs=pl.BlockSpec((TM, DIM), lambda j, te_r: (j, 0)),
    )
    return pl.pallas_call(
        _gemm_body,
        grid_spec=grid_spec,
        out_shape=jax.ShapeDtypeStruct((PADDED, DIM), jnp.float32),
    )(te, xs, W1, W3, W2)


THALF = TPW // 2


def _combine_body(yp_hbm, pos_hbm, w_hbm, y_hbm,
                  posh_v, w_v, rows_v, y_v, sem1):
    wid = lax.axis_index("s") * 2 + lax.axis_index("c")
    pbase = wid * PP

    pltpu.sync_copy(w_hbm.at[pl.ds(pbase, PP)], w_v)
    for h in range(2):
        pltpu.sync_copy(pos_hbm.at[pl.ds(pbase + h * 2 * THALF, 2 * THALF)],
                        posh_v)
        pltpu.async_copy(yp_hbm.at[posh_v], rows_v, sem1).wait()

        def tok_step(i, _, h=h):
            base_w = h * 2 * THALF
            w0 = plsc.load_gather(
                w_v, [jnp.zeros((16,), jnp.int32) + base_w + 2 * i])
            w1 = plsc.load_gather(
                w_v, [jnp.zeros((16,), jnp.int32) + base_w + 2 * i + 1])
            for j in range(DIM // 16):
                r0 = rows_v[2 * i, pl.ds(j * 16, 16)]
                r1 = rows_v[2 * i + 1, pl.ds(j * 16, 16)]
                y_v[i, pl.ds(j * 16, 16)] = w0 * r0 + w1 * r1
            return 0

        lax.fori_loop(0, THALF, tok_step, 0)
        pltpu.sync_copy(y_v, y_hbm.at[pl.ds(wid * TPW + h * THALF, THALF)])


def _combine(ypart, pos, w_all):
    mesh = plsc.VectorSubcoreMesh(core_axis_name="c", subcore_axis_name="s")
    return pl.kernel(
        _combine_body,
        mesh=mesh,
        out_type=jax.ShapeDtypeStruct((T, DIM), jnp.float32),
        scratch_types=[
            pltpu.VMEM((2 * THALF,), jnp.int32),
            pltpu.VMEM((PP,), jnp.float32),
            pltpu.VMEM((2 * THALF, DIM), jnp.float32),
            pltpu.VMEM((THALF, DIM), jnp.float32),
            pltpu.SemaphoreType.DMA,
        ],
        compiler_params=pltpu.CompilerParams(needs_layout_passes=False),
    )(ypart, pos, w_all)


def kernel(x, Wg, W1, W2, W3):
    b, s, d = x.shape
    flat = x.reshape(-1, d)
    wgp = jnp.zeros((LANES, DIM), jnp.float32).at[:E].set(Wg)

    idx_out, w_out = _gate(flat, wgp)
    e_all = idx_out[:, :K].reshape(-1)
    w_all = w_out[:, :K].reshape(-1)

    xs, pos, te = _route(e_all, flat)
    ypart = _gemm(te[:NTILES], xs, W1, W3, W2)
    y = _combine(ypart, pos, w_all)
    return y.reshape(b, s, d)


# speedup vs baseline: 1.1388x; 1.0340x over previous
import functools

import jax
import jax.numpy as jnp
from jax import lax
from jax.experimental import pallas as pl
from jax.experimental.pallas import tpu as pltpu
from jax.experimental.pallas import tpu_sc as plsc

DIM = 768
HID = 2048
E = 8
K = 2
T = 2048
PAIRS = T * K
TM = 512
TMSH = 9
NTILES = 16
PADDED = NTILES * TM
LANES = 128

NW = 32
PP = PAIRS // NW
TPW = T // NW
NCHUNK = PAIRS // 16
MYCH = PP // 16


def _gate_body(x_ref, wg_ref, idx_ref, w_ref):
    x = x_ref[...]
    lg = lax.dot_general(x, wg_ref[...], (((1,), (1,)), ((), ())),
                         preferred_element_type=jnp.float32)
    lane = lax.broadcasted_iota(jnp.int32, lg.shape, 1)
    neg = jnp.float32(-1e30)
    lg = jnp.where(lane < E, lg, neg)
    m1 = jnp.max(lg, axis=1, keepdims=True)
    i1 = jnp.min(jnp.where(lg == m1, lane, LANES), axis=1, keepdims=True)
    lg2 = jnp.where(lane == i1, neg, lg)
    m2 = jnp.max(lg2, axis=1, keepdims=True)
    i2 = jnp.min(jnp.where(lg2 == m2, lane, LANES), axis=1, keepdims=True)
    w_top = 1.0 / (1.0 + jnp.exp(m2 - m1))
    w_snd = 1.0 - w_top
    idx_ref[...] = jnp.where(lane == 0, i1, jnp.where(lane == 1, i2, 0))
    w_ref[...] = jnp.where(lane == 0, w_top, jnp.where(lane == 1, w_snd, 0.0))


def _gate(flat, wgp):
    return pl.pallas_call(
        _gate_body,
        grid=(1,),
        in_specs=[
            pl.BlockSpec((T, DIM), lambda i: (0, 0)),
            pl.BlockSpec((LANES, DIM), lambda i: (0, 0)),
        ],
        out_specs=[
            pl.BlockSpec((T, LANES), lambda i: (0, 0)),
            pl.BlockSpec((T, LANES), lambda i: (0, 0)),
        ],
        out_shape=[
            jax.ShapeDtypeStruct((T, LANES), jnp.int32),
            jax.ShapeDtypeStruct((T, LANES), jnp.float32),
        ],
    )(flat, wgp)


def _route_body(eall_hbm, x_hbm, xs_hbm, pos_hbm, te_hbm,
                eall_v, xrows_v, deste_v, desto_v, dest_v, cnt_v, gbuf_v,
                te_v, sem1, sem2):
    wid = lax.axis_index("s") * 2 + lax.axis_index("c")
    base = wid * PP
    lane = lax.broadcasted_iota(jnp.int32, (16,), 0)
    ones = jnp.ones((16,), jnp.int32)

    gcopy = pltpu.async_copy(x_hbm.at[pl.ds(wid * TPW, TPW)], xrows_v, sem1)
    pltpu.sync_copy(eall_hbm, eall_v)
    mych0 = wid * MYCH
    cnt_v[pl.ds(0, 16)] = jnp.zeros((16,), jnp.int32)

    def add_step(i, _):
        v = eall_v[pl.ds(i * 16, 16)]
        plsc.addupdate_scatter(cnt_v, [v], ones)
        return 0

    lax.fori_loop(0, mych0, add_step, 0)

    for c in range(MYCH):
        v = eall_v[pl.ds((mych0 + c) * 16, 16)]
        g = plsc.load_gather(cnt_v, [v])
        rank = jnp.zeros((16,), jnp.int32)
        for e in range(E):
            m = v == e
            cm = plsc.cumsum(m.astype(jnp.int32))
            rank = jnp.where(m, cm - 1, rank)
        dest_v[pl.ds(c * 16, 16)] = g + rank
        plsc.addupdate_scatter(cnt_v, [v], ones)

    lax.fori_loop(mych0 + MYCH, NCHUNK, add_step, 0)
    counts = cnt_v[pl.ds(0, 16)]

    pc = ((counts + (TM - 1)) >> TMSH) << TMSH
    cs = plsc.cumsum(pc)
    po = cs - pc

    cs_s = [jnp.sum(jnp.where(lane == e, cs, 0)) for e in range(E)]
    for c in range(4):
        tj = (lane + c * 16) * TM
        acc = jnp.zeros((16,), jnp.int32)
        for e in range(E):
            acc = acc + (tj >= cs_s[e]).astype(jnp.int32)
        te_v[pl.ds(c * 16, 16)] = acc

    @pl.when(wid == 0)
    def _():
        pltpu.sync_copy(te_v, te_hbm)

    gbuf_v[pl.ds(0, 16)] = po
    for c in range(MYCH):
        v = eall_v[pl.ds((mych0 + c) * 16, 16)]
        dest_v[pl.ds(c * 16, 16)] = (dest_v[pl.ds(c * 16, 16)]
                                     + plsc.load_gather(gbuf_v, [v]))

    for c2 in range(4):
        idx_e = c2 * 32 + 2 * lane
        deste_v[pl.ds(c2 * 16, 16)] = plsc.load_gather(dest_v, [idx_e])
        desto_v[pl.ds(c2 * 16, 16)] = plsc.load_gather(dest_v, [idx_e + 1])

    pltpu.sync_copy(dest_v, pos_hbm.at[pl.ds(base, PP)])
    gcopy.wait()
    s0 = pltpu.async_copy(xrows_v, xs_hbm.at[deste_v], sem2)
    s1 = pltpu.async_copy(xrows_v, xs_hbm.at[desto_v], sem1)
    s0.wait()
    s1.wait()


def _route(e_all, flat):
    mesh = plsc.VectorSubcoreMesh(core_axis_name="c", subcore_axis_name="s")
    return pl.kernel(
        _route_body,
        mesh=mesh,
        out_type=[
            jax.ShapeDtypeStruct((PADDED, DIM), jnp.float32),
            jax.ShapeDtypeStruct((PAIRS,), jnp.int32),
            jax.ShapeDtypeStruct((64,), jnp.int32),
        ],
        scratch_types=[
            pltpu.VMEM((PAIRS,), jnp.int32),
            pltpu.VMEM((TPW, DIM), jnp.float32),
            pltpu.VMEM((TPW,), jnp.int32),
            pltpu.VMEM((TPW,), jnp.int32),
            pltpu.VMEM((PP,), jnp.int32),
            pltpu.VMEM((16,), jnp.int32),
            pltpu.VMEM((32,), jnp.int32),
            pltpu.VMEM((64,), jnp.int32),
            pltpu.SemaphoreType.DMA,
            pltpu.SemaphoreType.DMA,
        ],
        compiler_params=pltpu.CompilerParams(needs_layout_passes=False),
    )(e_all, flat)


def _gemm_body(te_ref, x_ref, w1_ref, w3_ref, w2_ref, o_ref):
    j = pl.program_id(0)

    @pl.when(te_ref[j] < E)
    def _():
        x = x_ref[...]
        h1 = lax.dot_general(x, w1_ref[0], (((1,), (1,)), ((), ())),
                             preferred_element_type=jnp.float32)
        h3 = lax.dot_general(x, w3_ref[0], (((1,), (1,)), ((), ())),
                             preferred_element_type=jnp.float32)
        hid = (h1 * jax.nn.sigmoid(h1)) * h3
        o_ref[...] = lax.dot_general(hid, w2_ref[0], (((1,), (1,)), ((), ())),
                                     preferred_element_type=jnp.float32)


def _gemm(te, xs, W1, W3, W2):
    grid_spec = pltpu.PrefetchScalarGridSpec(
        num_scalar_prefetch=1,
        grid=(NTILES,),
        in_specs=[
            pl.BlockSpec((TM, DIM), lambda j, te_r: (j, 0)),
            pl.BlockSpec((1, HID, DIM),
                         lambda j, te_r: (jnp.minimum(te_r[j], E - 1), 0, 0)),
            pl.BlockSpec((1, HID, DIM),
                         lambda j, te_r: (jnp.minimum(te_r[j], E - 1), 0, 0)),
            pl.BlockSpec((1, DIM, HID),
                         lambda j, te_r: (jnp.minimum(te_r[j], E - 1), 0, 0)),
        ],
        out_specs=pl.BlockSpec((TM, DIM), lambda j, te_r: (j, 0)),
    )
    return pl.pallas_call(
        _gemm_body,
        grid_spec=grid_spec,
        out_shape=jax.ShapeDtypeStruct((PADDED, DIM), jnp.float32),
    )(te, xs, W1, W3, W2)


THALF = TPW // 2


def _combine_body(yp_hbm, pos_hbm, w_hbm, y_hbm,
                  pos_v, w_v, rows0_v, rows1_v, y_v, sem1, sem2):
    wid = lax.axis_index("s") * 2 + lax.axis_index("c")
    pbase = wid * PP

    pltpu.sync_copy(w_hbm.at[pl.ds(pbase, PP)], w_v)
    pltpu.sync_copy(pos_hbm.at[pl.ds(pbase, PP)], pos_v)
    cp0 = pltpu.async_copy(yp_hbm.at[pos_v.at[pl.ds(0, 2 * THALF)]],
                           rows0_v, sem1)
    cp1 = pltpu.async_copy(yp_hbm.at[pos_v.at[pl.ds(2 * THALF, 2 * THALF)]],
                           rows1_v, sem2)
    for h, (cp, rows_v) in enumerate(((cp0, rows0_v), (cp1, rows1_v))):
        cp.wait()

        def tok_step(i, _, h=h, rows_v=rows_v):
            base_w = h * 2 * THALF
            w0 = plsc.load_gather(
                w_v, [jnp.zeros((16,), jnp.int32) + base_w + 2 * i])
            w1 = plsc.load_gather(
                w_v, [jnp.zeros((16,), jnp.int32) + base_w + 2 * i + 1])
            for j in range(DIM // 16):
                r0 = rows_v[2 * i, pl.ds(j * 16, 16)]
                r1 = rows_v[2 * i + 1, pl.ds(j * 16, 16)]
                y_v[i, pl.ds(j * 16, 16)] = w0 * r0 + w1 * r1
            return 0

        lax.fori_loop(0, THALF, tok_step, 0)
        pltpu.sync_copy(y_v, y_hbm.at[pl.ds(wid * TPW + h * THALF, THALF)])


def _combine(ypart, pos, w_all):
    mesh = plsc.VectorSubcoreMesh(core_axis_name="c", subcore_axis_name="s")
    return pl.kernel(
        _combine_body,
        mesh=mesh,
        out_type=jax.ShapeDtypeStruct((T, DIM), jnp.float32),
        scratch_types=[
            pltpu.VMEM((PP,), jnp.int32),
            pltpu.VMEM((PP,), jnp.float32),
            pltpu.VMEM((2 * THALF, DIM), jnp.float32),
            pltpu.VMEM((2 * THALF, DIM), jnp.float32),
            pltpu.VMEM((THALF, DIM), jnp.float32),
            pltpu.SemaphoreType.DMA,
            pltpu.SemaphoreType.DMA,
        ],
        compiler_params=pltpu.CompilerParams(needs_layout_passes=False),
    )(ypart, pos, w_all)


def kernel(x, Wg, W1, W2, W3):
    b, s, d = x.shape
    flat = x.reshape(-1, d)
    wgp = jnp.zeros((LANES, DIM), jnp.float32).at[:E].set(Wg)

    idx_out, w_out = _gate(flat, wgp)
    e_all = idx_out[:, :K].reshape(-1)
    w_all = w_out[:, :K].reshape(-1)

    xs, pos, te = _route(e_all, flat)
    ypart = _gemm(te[:NTILES], xs, W1, W3, W2)
    y = _combine(ypart, pos, w_all)
    return y.reshape(b, s, d)
